# baseline (speedup 1.0000x reference)
import functools

import jax
import jax.numpy as jnp
from jax import lax
from jax.experimental import pallas as pl
from jax.experimental.pallas import tpu as pltpu
from jax.experimental.pallas import tpu_sc as plsc

N = 10000
E = 320000
D = 128
H = D // 2
NS = 16
EPS = E // NS
CH = 80
NCHUNK = EPS // CH
RPT = 624
REM = N - NS * RPT
ZR = 208


def _prep_body(x_ref, w_ref, b_ref, a_ref, a2_ref, bb_ref, bb2_ref):
    w1 = w_ref[0:D, :]
    w2 = w_ref[D : 2 * D, :]
    xb = x_ref[...]
    a = jnp.dot(xb, w1 - w2, preferred_element_type=jnp.float32) + b_ref[...]
    bb = jnp.dot(xb, w2, preferred_element_type=jnp.float32)
    a_ref[...] = a[:, 0:H].astype(jnp.bfloat16)
    a2_ref[...] = a[:, H:D].astype(jnp.bfloat16)
    bb_ref[...] = bb[:, 0:H].astype(jnp.bfloat16)
    bb2_ref[...] = bb[:, H:D].astype(jnp.bfloat16)


def _prep(x, w, b2d):
    bm = 1000
    half = jax.ShapeDtypeStruct((N, H), jnp.bfloat16)
    return pl.pallas_call(
        _prep_body,
        grid=(N // bm,),
        in_specs=[
            pl.BlockSpec((bm, D), lambda i: (i, 0)),
            pl.BlockSpec((2 * D, D), lambda i: (0, 0)),
            pl.BlockSpec((1, D), lambda i: (0, 0)),
        ],
        out_specs=[pl.BlockSpec((bm, H), lambda i: (i, 0))] * 4,
        out_shape=[half] * 4,
    )(x, w, b2d)


_MESH = plsc.VectorSubcoreMesh(core_axis_name="c", subcore_axis_name="s")


@functools.partial(
    pl.kernel,
    mesh=_MESH,
    compiler_params=pltpu.CompilerParams(
        use_tc_tiling_on_sc=False, needs_layout_passes=False
    ),
    out_type=[
        jax.ShapeDtypeStruct((2, N, H), jnp.float32),
        jax.ShapeDtypeStruct((2, N, 16), jnp.float32),
    ],
    scratch_types=[
        pltpu.VMEM((NCHUNK, CH), jnp.int32),
        pltpu.VMEM((NCHUNK, CH), jnp.int32),
        pltpu.VMEM((CH, H), jnp.bfloat16),
        pltpu.VMEM((CH, H), jnp.bfloat16),
        pltpu.VMEM((CH, H), jnp.float32),
        pltpu.VMEM((CH, 16), jnp.float32),
        pltpu.VMEM((ZR, H), jnp.float32),
        pltpu.VMEM((ZR, 16), jnp.float32),
        pltpu.VMEM_SHARED((N, H), jnp.float32),
        pltpu.VMEM_SHARED((N, 16), jnp.float32),
        pltpu.SemaphoreType.DMA,
    ],
)
def _sc_main(
    tgt_hbm, src_hbm, a0_hbm, a1_hbm, b0_hbm, b1_hbm,
    out_hbm, deg_hbm,
    tgt_v, src_v, a_v, b_v, m_v, ones_v, zrow_v, zdeg_v, acc_sh, deg_sh,
    sem_g,
):
    cid = lax.axis_index("c")
    sid = lax.axis_index("s")

    pltpu.sync_copy(tgt_hbm.at[sid], tgt_v)
    pltpu.sync_copy(src_hbm.at[sid], src_v)

    zero16 = jnp.zeros((16,), jnp.float32)
    one16 = jnp.ones((16,), jnp.float32)

    def _fill_zrow(i, carry):
        for g in range(H // 16):
            zrow_v[i, pl.ds(g * 16, 16)] = zero16
        zdeg_v[i, :] = zero16
        return carry

    lax.fori_loop(0, ZR, _fill_zrow, 0)

    def _fill_ones(i, carry):
        ones_v[i, :] = one16
        return carry

    lax.fori_loop(0, CH, _fill_ones, 0)

    for r in range(RPT // ZR):
        pltpu.sync_copy(zrow_v, acc_sh.at[pl.ds(sid * RPT + r * ZR, ZR)])
        pltpu.sync_copy(zdeg_v, deg_sh.at[pl.ds(sid * RPT + r * ZR, ZR)])

    @pl.when(sid == 15)
    def _zero_tail():
        pltpu.sync_copy(zrow_v.at[pl.ds(0, REM)], acc_sh.at[pl.ds(NS * RPT, REM)])
        pltpu.sync_copy(zdeg_v.at[pl.ds(0, REM)], deg_sh.at[pl.ds(NS * RPT, REM)])

    plsc.subcore_barrier()

    def _chunk(ci, carry):
        ti = tgt_v.at[ci]
        si = src_v.at[ci]

        @pl.when(cid == 0)
        def _gather0():
            pltpu.async_copy(a0_hbm.at[ti], a_v, sem_g)
            pltpu.async_copy(b0_hbm.at[si], b_v, sem_g)

        @pl.when(cid == 1)
        def _gather1():
            pltpu.async_copy(a1_hbm.at[ti], a_v, sem_g)
            pltpu.async_copy(b1_hbm.at[si], b_v, sem_g)

        pltpu.make_async_copy(a0_hbm.at[ti], a_v, sem_g).wait()
        pltpu.make_async_copy(b0_hbm.at[si], b_v, sem_g).wait()

        def _edges(e, c2):
            for g in range(H // 32):
                av = a_v[e, pl.ds(g * 32, 32)]
                bv = b_v[e, pl.ds(g * 32, 32)]
                alo, ahi = plsc.unpack(av, format=plsc.PackFormat.INTERLEAVED)
                blo, bhi = plsc.unpack(bv, format=plsc.PackFormat.INTERLEAVED)
                m_v[e, pl.ds(g * 32, 16)] = jnp.maximum(alo + blo, 0.0)
                m_v[e, pl.ds(g * 32 + 16, 16)] = jnp.maximum(ahi + bhi, 0.0)
            return c2

        lax.fori_loop(0, CH, _edges, 0)

        pltpu.sync_copy(m_v, acc_sh.at[ti], add=True)
        pltpu.sync_copy(ones_v, deg_sh.at[ti], add=True)
        return carry

    lax.fori_loop(0, NCHUNK, _chunk, 0)
    plsc.subcore_barrier()

    pltpu.sync_copy(acc_sh.at[pl.ds(sid * RPT, RPT)],
                    out_hbm.at[cid, pl.ds(sid * RPT, RPT)])
    pltpu.sync_copy(deg_sh.at[pl.ds(sid * RPT, RPT)],
                    deg_hbm.at[cid, pl.ds(sid * RPT, RPT)])

    @pl.when(sid == 15)
    def _write_tail():
        pltpu.sync_copy(acc_sh.at[pl.ds(NS * RPT, REM)],
                        out_hbm.at[cid, pl.ds(NS * RPT, REM)])
        pltpu.sync_copy(deg_sh.at[pl.ds(NS * RPT, REM)],
                        deg_hbm.at[cid, pl.ds(NS * RPT, REM)])


def _fin_body(acc_ref, deg_ref, o_ref):
    d0 = deg_ref[0, :, 0:1] + 1e-8
    d1 = deg_ref[1, :, 0:1] + 1e-8
    o_ref[:, 0:H] = acc_ref[0, :, :] / d0
    o_ref[:, H:D] = acc_ref[1, :, :] / d1


def _finalize(acc, deg):
    bm = 1000
    return pl.pallas_call(
        _fin_body,
        grid=(N // bm,),
        in_specs=[
            pl.BlockSpec((2, bm, H), lambda i: (0, i, 0)),
            pl.BlockSpec((2, bm, 16), lambda i: (0, i, 0)),
        ],
        out_specs=pl.BlockSpec((bm, D), lambda i: (i, 0)),
        out_shape=jax.ShapeDtypeStruct((N, D), jnp.float32),
    )(acc, deg)


def _interleave_cols(t):
    return t.reshape(N, H // 32, 2, 16).transpose(0, 1, 3, 2).reshape(N, H)


def kernel(x, W, b, k, nn_index):
    tabs = [_interleave_cols(t) for t in _prep(x, W, b.reshape(1, D))]
    src = nn_index[0].astype(jnp.int32).reshape(NS, NCHUNK, CH)
    tgt = nn_index[1].astype(jnp.int32).reshape(NS, NCHUNK, CH)
    acc, deg = _sc_main(tgt, src, *tabs)
    return _finalize(acc, deg)


# speedup vs baseline: 2.3756x; 2.3756x over previous
import functools

import jax
import jax.numpy as jnp
from jax import lax
from jax.experimental import pallas as pl
from jax.experimental.pallas import tpu as pltpu
from jax.experimental.pallas import tpu_sc as plsc

N = 10000
E = 320000
D = 128
H = D // 2
NS = 16
EPS = E // NS
CH = 80
NCHUNK = EPS // CH
RPT = 624
REM = N - NS * RPT
ZR = 104


def _prep_body(x_ref, w_ref, b_ref, a_ref, a2_ref, bb_ref, bb2_ref):
    w1 = w_ref[0:D, :]
    w2 = w_ref[D : 2 * D, :]
    xb = x_ref[...]
    a = jnp.dot(xb, w1 - w2, preferred_element_type=jnp.float32) + b_ref[...]
    bb = jnp.dot(xb, w2, preferred_element_type=jnp.float32)
    a_ref[...] = a[:, 0:H]
    a2_ref[...] = a[:, H:D]
    bb_ref[...] = bb[:, 0:H]
    bb2_ref[...] = bb[:, H:D]


def _prep(x, w, b2d):
    bm = 1000
    half = jax.ShapeDtypeStruct((N, H), jnp.float32)
    return pl.pallas_call(
        _prep_body,
        grid=(N // bm,),
        in_specs=[
            pl.BlockSpec((bm, D), lambda i: (i, 0)),
            pl.BlockSpec((2 * D, D), lambda i: (0, 0)),
            pl.BlockSpec((1, D), lambda i: (0, 0)),
        ],
        out_specs=[pl.BlockSpec((bm, H), lambda i: (i, 0))] * 4,
        out_shape=[half] * 4,
    )(x, w, b2d)


_MESH = plsc.VectorSubcoreMesh(core_axis_name="c", subcore_axis_name="s")


@functools.partial(
    pl.kernel,
    mesh=_MESH,
    compiler_params=pltpu.CompilerParams(use_tc_tiling_on_sc=False),
    out_type=[
        jax.ShapeDtypeStruct((2, N, H), jnp.float32),
        jax.ShapeDtypeStruct((2, N, 16), jnp.float32),
    ],
    scratch_types=[
        pltpu.VMEM((NCHUNK, CH), jnp.int32),
        pltpu.VMEM((NCHUNK, CH), jnp.int32),
        pltpu.VMEM((CH, H), jnp.float32),
        pltpu.VMEM((CH, H), jnp.float32),
        pltpu.VMEM((CH, H), jnp.float32),
        pltpu.VMEM((CH, H), jnp.float32),
        pltpu.VMEM((CH, H), jnp.float32),
        pltpu.VMEM((CH, H), jnp.float32),
        pltpu.VMEM((CH, 16), jnp.float32),
        pltpu.VMEM((ZR, H), jnp.float32),
        pltpu.VMEM((ZR, 16), jnp.float32),
        pltpu.VMEM_SHARED((N, H), jnp.float32),
        pltpu.VMEM_SHARED((N, 16), jnp.float32),
        pltpu.SemaphoreType.DMA,
        pltpu.SemaphoreType.DMA,
        pltpu.SemaphoreType.DMA,
        pltpu.SemaphoreType.DMA,
    ],
)
def _sc_main(
    tgt_hbm, src_hbm, a0_hbm, a1_hbm, b0_hbm, b1_hbm,
    out_hbm, deg_hbm,
    tgt_v, src_v, a_v0, a_v1, b_v0, b_v1, m_v0, m_v1, ones_v, zrow_v, zdeg_v,
    acc_sh, deg_sh, sem_g0, sem_g1, sem_s0, sem_s1,
):
    cid = lax.axis_index("c")
    sid = lax.axis_index("s")

    pltpu.sync_copy(tgt_hbm.at[sid], tgt_v)
    pltpu.sync_copy(src_hbm.at[sid], src_v)

    zero16 = jnp.zeros((16,), jnp.float32)
    one16 = jnp.ones((16,), jnp.float32)

    def _fill_zrow(i, carry):
        for g in range(H // 16):
            zrow_v[i, pl.ds(g * 16, 16)] = zero16
        zdeg_v[i, :] = zero16
        return carry

    lax.fori_loop(0, ZR, _fill_zrow, 0)

    def _fill_ones(i, carry):
        ones_v[i, :] = one16
        return carry

    lax.fori_loop(0, CH, _fill_ones, 0)

    for r in range(RPT // ZR):
        pltpu.sync_copy(zrow_v, acc_sh.at[pl.ds(sid * RPT + r * ZR, ZR)])
        pltpu.sync_copy(zdeg_v, deg_sh.at[pl.ds(sid * RPT + r * ZR, ZR)])

    @pl.when(sid == 15)
    def _zero_tail():
        pltpu.sync_copy(zrow_v.at[pl.ds(0, REM)], acc_sh.at[pl.ds(NS * RPT, REM)])
        pltpu.sync_copy(zdeg_v.at[pl.ds(0, REM)], deg_sh.at[pl.ds(NS * RPT, REM)])

    plsc.subcore_barrier()

    def _issue(ci, a_v, b_v, sem):
        @pl.when(cid == 0)
        def _issue0():
            pltpu.async_copy(a0_hbm.at[tgt_v.at[ci]], a_v, sem)
            pltpu.async_copy(b0_hbm.at[src_v.at[ci]], b_v, sem)

        @pl.when(cid == 1)
        def _issue1():
            pltpu.async_copy(a1_hbm.at[tgt_v.at[ci]], a_v, sem)
            pltpu.async_copy(b1_hbm.at[src_v.at[ci]], b_v, sem)

    _issue(0, a_v0, b_v0, sem_g0)
    _issue(1, a_v1, b_v1, sem_g1)

    slots = ((a_v0, b_v0, m_v0, sem_g0, sem_s0), (a_v1, b_v1, m_v1, sem_g1, sem_s1))

    def _pair(p, carry):
        for b, (a_v, b_v, m_v, sem_g, sem_s) in enumerate(slots):
            ci = 2 * p + b
            ti = tgt_v.at[ci]
            pltpu.make_async_copy(a0_hbm.at[ti], a_v, sem_g).wait()
            pltpu.make_async_copy(b0_hbm.at[ti], b_v, sem_g).wait()

            @pl.when(ci >= 2)
            def _drain_scatter():
                pltpu.make_async_copy(m_v, acc_sh.at[ti], sem_s).wait()

            def _edges(e, c2):
                for g in range(H // 16):
                    av = a_v[e, pl.ds(g * 16, 16)]
                    bv = b_v[e, pl.ds(g * 16, 16)]
                    m_v[e, pl.ds(g * 16, 16)] = jnp.maximum(av + bv, 0.0)
                return c2

            lax.fori_loop(0, CH, _edges, 0)

            pltpu.async_copy(m_v, acc_sh.at[ti], sem_s, add=True)
            pltpu.sync_copy(ones_v, deg_sh.at[ti], add=True)

            @pl.when(ci + 2 < NCHUNK)
            def _refill():
                _issue(ci + 2, a_v, b_v, sem_g)
        return carry

    lax.fori_loop(0, NCHUNK // 2, _pair, 0)

    pltpu.make_async_copy(m_v0, acc_sh.at[tgt_v.at[0]], sem_s0).wait()
    pltpu.make_async_copy(m_v1, acc_sh.at[tgt_v.at[0]], sem_s1).wait()
    plsc.subcore_barrier()

    pltpu.sync_copy(acc_sh.at[pl.ds(sid * RPT, RPT)],
                    out_hbm.at[cid, pl.ds(sid * RPT, RPT)])
    pltpu.sync_copy(deg_sh.at[pl.ds(sid * RPT, RPT)],
                    deg_hbm.at[cid, pl.ds(sid * RPT, RPT)])

    @pl.when(sid == 15)
    def _write_tail():
        pltpu.sync_copy(acc_sh.at[pl.ds(NS * RPT, REM)],
                        out_hbm.at[cid, pl.ds(NS * RPT, REM)])
        pltpu.sync_copy(deg_sh.at[pl.ds(NS * RPT, REM)],
                        deg_hbm.at[cid, pl.ds(NS * RPT, REM)])


def _fin_body(acc_ref, deg_ref, o_ref):
    d0 = deg_ref[0, :, 0:1] + 1e-8
    d1 = deg_ref[1, :, 0:1] + 1e-8
    o_ref[:, 0:H] = acc_ref[0, :, :] / d0
    o_ref[:, H:D] = acc_ref[1, :, :] / d1


def _finalize(acc, deg):
    bm = 1000
    return pl.pallas_call(
        _fin_body,
        grid=(N // bm,),
        in_specs=[
            pl.BlockSpec((2, bm, H), lambda i: (0, i, 0)),
            pl.BlockSpec((2, bm, 16), lambda i: (0, i, 0)),
        ],
        out_specs=pl.BlockSpec((bm, D), lambda i: (i, 0)),
        out_shape=jax.ShapeDtypeStruct((N, D), jnp.float32),
    )(acc, deg)


def kernel(x, W, b, k, nn_index):
    tabs = _prep(x, W, b.reshape(1, D))
    src = nn_index[0].astype(jnp.int32).reshape(NS, NCHUNK, CH)
    tgt = nn_index[1].astype(jnp.int32).reshape(NS, NCHUNK, CH)
    acc, deg = _sc_main(tgt, src, *tabs)
    return _finalize(acc, deg)


# speedup vs baseline: 2.4442x; 1.0289x over previous
import functools

import jax
import jax.numpy as jnp
from jax import lax
from jax.experimental import pallas as pl
from jax.experimental.pallas import tpu as pltpu
from jax.experimental.pallas import tpu_sc as plsc

N = 10000
E = 320000
D = 128
H = D // 2
NS = 16
EPS = E // NS
CH = 80
NCHUNK = EPS // CH
RPT = 624
REM = N - NS * RPT
ZR = 104


def _prep_body(x_ref, w_ref, b_ref, a_ref, a2_ref, bb_ref, bb2_ref):
    w1 = w_ref[0:D, :]
    w2 = w_ref[D : 2 * D, :]
    xb = x_ref[...]
    a = jnp.dot(xb, w1 - w2, preferred_element_type=jnp.float32) + b_ref[...]
    bb = jnp.dot(xb, w2, preferred_element_type=jnp.float32)
    a_ref[...] = a[:, 0:H]
    a2_ref[...] = a[:, H:D]
    bb_ref[...] = bb[:, 0:H]
    bb2_ref[...] = bb[:, H:D]


def _prep(x, w, b2d):
    bm = 1000
    half = jax.ShapeDtypeStruct((N, H), jnp.float32)
    return pl.pallas_call(
        _prep_body,
        grid=(N // bm,),
        in_specs=[
            pl.BlockSpec((bm, D), lambda i: (i, 0)),
            pl.BlockSpec((2 * D, D), lambda i: (0, 0)),
            pl.BlockSpec((1, D), lambda i: (0, 0)),
        ],
        out_specs=[pl.BlockSpec((bm, H), lambda i: (i, 0))] * 4,
        out_shape=[half] * 4,
    )(x, w, b2d)


_MESH = plsc.VectorSubcoreMesh(core_axis_name="c", subcore_axis_name="s")


@functools.partial(
    pl.kernel,
    mesh=_MESH,
    compiler_params=pltpu.CompilerParams(use_tc_tiling_on_sc=False),
    out_type=[
        jax.ShapeDtypeStruct((2, N, H), jnp.float32),
        jax.ShapeDtypeStruct((2, N, 16), jnp.float32),
    ],
    scratch_types=[
        pltpu.VMEM((NCHUNK, CH), jnp.int32),
        pltpu.VMEM((NCHUNK, CH), jnp.int32),
        pltpu.VMEM((CH, H), jnp.float32),
        pltpu.VMEM((CH, H), jnp.float32),
        pltpu.VMEM((CH, H), jnp.float32),
        pltpu.VMEM((CH, H), jnp.float32),
        pltpu.VMEM((CH, H), jnp.float32),
        pltpu.VMEM((CH, H), jnp.float32),
        pltpu.VMEM((CH, 16), jnp.float32),
        pltpu.VMEM((ZR, H), jnp.float32),
        pltpu.VMEM((ZR, 16), jnp.float32),
        pltpu.VMEM_SHARED((N, H), jnp.float32),
        pltpu.VMEM_SHARED((N, 16), jnp.float32),
        pltpu.SemaphoreType.DMA,
        pltpu.SemaphoreType.DMA,
        pltpu.SemaphoreType.DMA,
        pltpu.SemaphoreType.DMA,
        pltpu.SemaphoreType.DMA,
        pltpu.SemaphoreType.DMA,
    ],
)
def _sc_main(
    tgt_hbm, src_hbm, a0_hbm, a1_hbm, b0_hbm, b1_hbm,
    out_hbm, deg_hbm,
    tgt_v, src_v, a_v0, a_v1, b_v0, b_v1, m_v0, m_v1, ones_v, zrow_v, zdeg_v,
    acc_sh, deg_sh, sem_g0, sem_g1, sem_s0, sem_s1, sem_o0, sem_o1,
):
    cid = lax.axis_index("c")
    sid = lax.axis_index("s")

    pltpu.sync_copy(tgt_hbm.at[sid], tgt_v)
    pltpu.sync_copy(src_hbm.at[sid], src_v)

    zero16 = jnp.zeros((16,), jnp.float32)
    one16 = jnp.ones((16,), jnp.float32)

    def _fill_zrow(i, carry):
        for g in range(H // 16):
            zrow_v[i, pl.ds(g * 16, 16)] = zero16
        zdeg_v[i, :] = zero16
        return carry

    lax.fori_loop(0, ZR, _fill_zrow, 0)

    def _fill_ones(i, carry):
        ones_v[i, :] = one16
        return carry

    lax.fori_loop(0, CH, _fill_ones, 0)

    for r in range(RPT // ZR):
        pltpu.sync_copy(zrow_v, acc_sh.at[pl.ds(sid * RPT + r * ZR, ZR)])
        pltpu.sync_copy(zdeg_v, deg_sh.at[pl.ds(sid * RPT + r * ZR, ZR)])

    @pl.when(sid == 15)
    def _zero_tail():
        pltpu.sync_copy(zrow_v.at[pl.ds(0, REM)], acc_sh.at[pl.ds(NS * RPT, REM)])
        pltpu.sync_copy(zdeg_v.at[pl.ds(0, REM)], deg_sh.at[pl.ds(NS * RPT, REM)])

    plsc.subcore_barrier()

    def _issue(ci, a_v, b_v, sem):
        @pl.when(cid == 0)
        def _issue0():
            pltpu.async_copy(a0_hbm.at[tgt_v.at[ci]], a_v, sem)
            pltpu.async_copy(b0_hbm.at[src_v.at[ci]], b_v, sem)

        @pl.when(cid == 1)
        def _issue1():
            pltpu.async_copy(a1_hbm.at[tgt_v.at[ci]], a_v, sem)
            pltpu.async_copy(b1_hbm.at[src_v.at[ci]], b_v, sem)

    _issue(0, a_v0, b_v0, sem_g0)
    _issue(1, a_v1, b_v1, sem_g1)

    slots = ((a_v0, b_v0, m_v0, sem_g0, sem_s0, sem_o0),
             (a_v1, b_v1, m_v1, sem_g1, sem_s1, sem_o1))
    EU = 5

    def _pair(p, carry):
        for b, (a_v, b_v, m_v, sem_g, sem_s, sem_o) in enumerate(slots):
            ci = 2 * p + b
            ti = tgt_v.at[ci]
            pltpu.make_async_copy(a0_hbm.at[ti], a_v, sem_g).wait()
            pltpu.make_async_copy(b0_hbm.at[ti], b_v, sem_g).wait()

            @pl.when(ci >= 2)
            def _drain_scatter():
                pltpu.make_async_copy(m_v, acc_sh.at[ti], sem_s).wait()
                pltpu.make_async_copy(ones_v, deg_sh.at[ti], sem_o).wait()

            def _edges(u, c2):
                for de in range(EU):
                    e = u * EU + de
                    for g in range(H // 16):
                        av = a_v[e, pl.ds(g * 16, 16)]
                        bv = b_v[e, pl.ds(g * 16, 16)]
                        m_v[e, pl.ds(g * 16, 16)] = jnp.maximum(av + bv, 0.0)
                return c2

            lax.fori_loop(0, CH // EU, _edges, 0)

            pltpu.async_copy(m_v, acc_sh.at[ti], sem_s, add=True)
            pltpu.async_copy(ones_v, deg_sh.at[ti], sem_o, add=True)

            @pl.when(ci + 2 < NCHUNK)
            def _refill():
                _issue(ci + 2, a_v, b_v, sem_g)
        return carry

    lax.fori_loop(0, NCHUNK // 2, _pair, 0)

    pltpu.make_async_copy(m_v0, acc_sh.at[tgt_v.at[0]], sem_s0).wait()
    pltpu.make_async_copy(m_v1, acc_sh.at[tgt_v.at[0]], sem_s1).wait()
    pltpu.make_async_copy(ones_v, deg_sh.at[tgt_v.at[0]], sem_o0).wait()
    pltpu.make_async_copy(ones_v, deg_sh.at[tgt_v.at[0]], sem_o1).wait()
    plsc.subcore_barrier()

    pltpu.sync_copy(acc_sh.at[pl.ds(sid * RPT, RPT)],
                    out_hbm.at[cid, pl.ds(sid * RPT, RPT)])
    pltpu.sync_copy(deg_sh.at[pl.ds(sid * RPT, RPT)],
                    deg_hbm.at[cid, pl.ds(sid * RPT, RPT)])

    @pl.when(sid == 15)
    def _write_tail():
        pltpu.sync_copy(acc_sh.at[pl.ds(NS * RPT, REM)],
                        out_hbm.at[cid, pl.ds(NS * RPT, REM)])
        pltpu.sync_copy(deg_sh.at[pl.ds(NS * RPT, REM)],
                        deg_hbm.at[cid, pl.ds(NS * RPT, REM)])


def _fin_body(acc_ref, deg_ref, o_ref):
    d0 = deg_ref[0, :, 0:1] + 1e-8
    d1 = deg_ref[1, :, 0:1] + 1e-8
    o_ref[:, 0:H] = acc_ref[0, :, :] / d0
    o_ref[:, H:D] = acc_ref[1, :, :] / d1


def _finalize(acc, deg):
    bm = 1000
    return pl.pallas_call(
        _fin_body,
        grid=(N // bm,),
        in_specs=[
            pl.BlockSpec((2, bm, H), lambda i: (0, i, 0)),
            pl.BlockSpec((2, bm, 16), lambda i: (0, i, 0)),
        ],
        out_specs=pl.BlockSpec((bm, D), lambda i: (i, 0)),
        out_shape=jax.ShapeDtypeStruct((N, D), jnp.float32),
    )(acc, deg)


def kernel(x, W, b, k, nn_index):
    tabs = _prep(x, W, b.reshape(1, D))
    src = nn_index[0].astype(jnp.int32).reshape(NS, NCHUNK, CH)
    tgt = nn_index[1].astype(jnp.int32).reshape(NS, NCHUNK, CH)
    acc, deg = _sc_main(tgt, src, *tabs)
    return _finalize(acc, deg)


# speedup vs baseline: 2.5993x; 1.0635x over previous
import functools

import jax
import jax.numpy as jnp
from jax import lax
from jax.experimental import pallas as pl
from jax.experimental.pallas import tpu as pltpu
from jax.experimental.pallas import tpu_sc as plsc

N = 10000
E = 320000
D = 128
H = D // 2
NS = 16
EPS = E // NS
CH = 80
NCHUNK = EPS // CH
RPT = 624
REM = N - NS * RPT
ZR = 104


def _prep_body(x_ref, w_ref, b_ref, a_ref, a2_ref, bb_ref, bb2_ref):
    w1 = w_ref[0:D, :]
    w2 = w_ref[D : 2 * D, :]
    xb = x_ref[...]
    a = jnp.dot(xb, w1 - w2, preferred_element_type=jnp.float32) + b_ref[...]
    bb = jnp.dot(xb, w2, preferred_element_type=jnp.float32)
    a_ref[...] = a[:, 0:H]
    a2_ref[...] = a[:, H:D]
    bb_ref[...] = bb[:, 0:H]
    bb2_ref[...] = bb[:, H:D]


def _prep(x, w, b2d):
    bm = 1000
    half = jax.ShapeDtypeStruct((N, H), jnp.float32)
    return pl.pallas_call(
        _prep_body,
        grid=(N // bm,),
        in_specs=[
            pl.BlockSpec((bm, D), lambda i: (i, 0)),
            pl.BlockSpec((2 * D, D), lambda i: (0, 0)),
            pl.BlockSpec((1, D), lambda i: (0, 0)),
        ],
        out_specs=[pl.BlockSpec((bm, H), lambda i: (i, 0))] * 4,
        out_shape=[half] * 4,
    )(x, w, b2d)


_MESH = plsc.VectorSubcoreMesh(core_axis_name="c", subcore_axis_name="s")


@functools.partial(
    pl.kernel,
    mesh=_MESH,
    compiler_params=pltpu.CompilerParams(use_tc_tiling_on_sc=False),
    out_type=jax.ShapeDtypeStruct((N, D), jnp.float32),
    scratch_types=[
        pltpu.VMEM((NCHUNK, CH), jnp.int32),
        pltpu.VMEM((NCHUNK, CH), jnp.int32),
        pltpu.VMEM((CH, H), jnp.float32),
        pltpu.VMEM((CH, H), jnp.float32),
        pltpu.VMEM((CH, H), jnp.float32),
        pltpu.VMEM((CH, H), jnp.float32),
        pltpu.VMEM((CH, H), jnp.float32),
        pltpu.VMEM((CH, H), jnp.float32),
        pltpu.VMEM((CH, 16), jnp.float32),
        pltpu.VMEM((ZR, H), jnp.float32),
        pltpu.VMEM((ZR, 16), jnp.float32),
        pltpu.VMEM_SHARED((N, H), jnp.float32),
        pltpu.VMEM_SHARED((N, 16), jnp.float32),
        pltpu.SemaphoreType.DMA,
        pltpu.SemaphoreType.DMA,
        pltpu.SemaphoreType.DMA,
        pltpu.SemaphoreType.DMA,
        pltpu.SemaphoreType.DMA,
        pltpu.SemaphoreType.DMA,
    ],
)
def _sc_main(
    tgt_hbm, src_hbm, a0_hbm, a1_hbm, b0_hbm, b1_hbm,
    out_hbm,
    tgt_v, src_v, a_v0, a_v1, b_v0, b_v1, m_v0, m_v1, ones_v, zrow_v, zdeg_v,
    acc_sh, deg_sh, sem_g0, sem_g1, sem_s0, sem_s1, sem_o0, sem_o1,
):
    cid = lax.axis_index("c")
    sid = lax.axis_index("s")

    pltpu.sync_copy(tgt_hbm.at[sid], tgt_v)
    pltpu.sync_copy(src_hbm.at[sid], src_v)

    zero16 = jnp.zeros((16,), jnp.float32)
    one16 = jnp.ones((16,), jnp.float32)

    def _fill_zrow(i, carry):
        for g in range(H // 16):
            zrow_v[i, pl.ds(g * 16, 16)] = zero16
        zdeg_v[i, :] = zero16
        return carry

    lax.fori_loop(0, ZR, _fill_zrow, 0)

    def _fill_ones(i, carry):
        ones_v[i, :] = one16
        return carry

    lax.fori_loop(0, CH, _fill_ones, 0)

    for r in range(RPT // ZR):
        pltpu.sync_copy(zrow_v, acc_sh.at[pl.ds(sid * RPT + r * ZR, ZR)])
        pltpu.sync_copy(zdeg_v, deg_sh.at[pl.ds(sid * RPT + r * ZR, ZR)])

    @pl.when(sid == 15)
    def _zero_tail():
        pltpu.sync_copy(zrow_v.at[pl.ds(0, REM)], acc_sh.at[pl.ds(NS * RPT, REM)])
        pltpu.sync_copy(zdeg_v.at[pl.ds(0, REM)], deg_sh.at[pl.ds(NS * RPT, REM)])

    plsc.subcore_barrier()

    def _issue(ci, a_v, b_v, sem):
        @pl.when(cid == 0)
        def _issue0():
            pltpu.async_copy(a0_hbm.at[tgt_v.at[ci]], a_v, sem)
            pltpu.async_copy(b0_hbm.at[src_v.at[ci]], b_v, sem)

        @pl.when(cid == 1)
        def _issue1():
            pltpu.async_copy(a1_hbm.at[tgt_v.at[ci]], a_v, sem)
            pltpu.async_copy(b1_hbm.at[src_v.at[ci]], b_v, sem)

    _issue(0, a_v0, b_v0, sem_g0)
    _issue(1, a_v1, b_v1, sem_g1)

    slots = ((a_v0, b_v0, m_v0, sem_g0, sem_s0, sem_o0),
             (a_v1, b_v1, m_v1, sem_g1, sem_s1, sem_o1))
    EU = 5

    def _pair(p, carry):
        for b, (a_v, b_v, m_v, sem_g, sem_s, sem_o) in enumerate(slots):
            ci = 2 * p + b
            ti = tgt_v.at[ci]
            pltpu.make_async_copy(a0_hbm.at[ti], a_v, sem_g).wait()
            pltpu.make_async_copy(b0_hbm.at[ti], b_v, sem_g).wait()

            @pl.when(ci >= 2)
            def _drain_scatter():
                pltpu.make_async_copy(m_v, acc_sh.at[ti], sem_s).wait()
                pltpu.make_async_copy(ones_v, deg_sh.at[ti], sem_o).wait()

            def _edges(u, c2):
                for de in range(EU):
                    e = u * EU + de
                    for g in range(H // 16):
                        av = a_v[e, pl.ds(g * 16, 16)]
                        bv = b_v[e, pl.ds(g * 16, 16)]
                        m_v[e, pl.ds(g * 16, 16)] = jnp.maximum(av + bv, 0.0)
                return c2

            lax.fori_loop(0, CH // EU, _edges, 0)

            pltpu.async_copy(m_v, acc_sh.at[ti], sem_s, add=True)
            pltpu.async_copy(ones_v, deg_sh.at[ti], sem_o, add=True)

            @pl.when(ci + 2 < NCHUNK)
            def _refill():
                _issue(ci + 2, a_v, b_v, sem_g)
        return carry

    lax.fori_loop(0, NCHUNK // 2, _pair, 0)

    pltpu.make_async_copy(m_v0, acc_sh.at[tgt_v.at[0]], sem_s0).wait()
    pltpu.make_async_copy(m_v1, acc_sh.at[tgt_v.at[0]], sem_s1).wait()
    pltpu.make_async_copy(ones_v, deg_sh.at[tgt_v.at[0]], sem_o0).wait()
    pltpu.make_async_copy(ones_v, deg_sh.at[tgt_v.at[0]], sem_o1).wait()
    plsc.subcore_barrier()

    def _fin_rows(base, nrows):
        pltpu.sync_copy(acc_sh.at[pl.ds(base, nrows)],
                        zrow_v.at[pl.ds(0, nrows)])
        pltpu.sync_copy(deg_sh.at[pl.ds(base, nrows)],
                        zdeg_v.at[pl.ds(0, nrows)])

        def _div_row(r, carry):
            dvi = 1.0 / (zdeg_v[r, :] + 1e-8)
            for g in range(H // 16):
                zrow_v[r, pl.ds(g * 16, 16)] = zrow_v[r, pl.ds(g * 16, 16)] * dvi
            return carry

        lax.fori_loop(0, nrows, _div_row, 0)
        pltpu.sync_copy(zrow_v.at[pl.ds(0, nrows)],
                        out_hbm.at[pl.ds(base, nrows), pl.ds(cid * H, H)])

    for s in range(RPT // ZR):
        _fin_rows(sid * RPT + s * ZR, ZR)

    @pl.when(sid == 15)
    def _fin_tail():
        _fin_rows(NS * RPT, REM)


def kernel(x, W, b, k, nn_index):
    tabs = _prep(x, W, b.reshape(1, D))
    src = nn_index[0].astype(jnp.int32).reshape(NS, NCHUNK, CH)
    tgt = nn_index[1].astype(jnp.int32).reshape(NS, NCHUNK, CH)
    return _sc_main(tgt, src, *tabs)
